# baseline (speedup 1.0000x reference)
import functools

import jax
import jax.numpy as jnp
from jax import lax
from jax.experimental import pallas as pl
from jax.experimental.pallas import tpu as pltpu
from jax.experimental.pallas import tpu_sc as plsc

LANES = 16
PADW = 128
BSTRIPE = 128
NBUF = 2


def _tta_kernel(n_batch, hist, dim, num_cores, num_subcores):
    n_workers = num_cores * num_subcores
    assert n_batch % (BSTRIPE * n_workers) == 0
    n_groups = hist // NBUF
    mesh = plsc.VectorSubcoreMesh(core_axis_name="c", subcore_axis_name="s")

    @functools.partial(
        pl.kernel,
        mesh=mesh,
        out_type=jax.ShapeDtypeStruct((hist, dim, n_batch), jnp.float32),
        compiler_params=pltpu.CompilerParams(needs_layout_passes=False),
        scratch_types=(
            [pltpu.VMEM((1, BSTRIPE), jnp.int32)] * NBUF
            + [pltpu.VMEM((1, dim, BSTRIPE), jnp.float32)] * NBUF
            + [pltpu.VMEM((BSTRIPE, PADW), jnp.float32)] * NBUF
            + [pltpu.SemaphoreType.DMA] * (3 * NBUF)
        ),
    )
    def k(x_hbm, idx_hbm, tab_hbm, out_hbm, *scr):
        idx_v = scr[0:NBUF]
        xb_v = scr[NBUF : 2 * NBUF]
        gb_v = scr[2 * NBUF : 3 * NBUF]
        in_sem = scr[3 * NBUF : 4 * NBUF]
        g_sem = scr[4 * NBUF : 5 * NBUF]
        out_sem = scr[5 * NBUF : 6 * NBUF]

        wid = lax.axis_index("s") * num_cores + lax.axis_index("c")
        b0 = wid * BSTRIPE
        iota16 = lax.iota(jnp.int32, LANES)

        def fire_in(b, h):
            pltpu.async_copy(
                idx_hbm.at[pl.ds(h, 1), pl.ds(b0, BSTRIPE)], idx_v[b], in_sem[b]
            )
            pltpu.async_copy(
                x_hbm.at[pl.ds(h, 1), :, pl.ds(b0, BSTRIPE)], xb_v[b], in_sem[b]
            )

        def wait_in(b):
            pltpu.make_async_copy(
                idx_hbm.at[pl.ds(0, 1), pl.ds(0, BSTRIPE)], idx_v[b], in_sem[b]
            ).wait()
            pltpu.make_async_copy(
                x_hbm.at[pl.ds(0, 1), :, pl.ds(0, BSTRIPE)], xb_v[b], in_sem[b]
            ).wait()

        def wait_out(b):
            pltpu.make_async_copy(
                xb_v[b], out_hbm.at[pl.ds(0, 1), :, pl.ds(0, BSTRIPE)], out_sem[b]
            ).wait()

        for b in range(NBUF):
            fire_in(b, b)

        def group_body(g, carry):
            h0 = g * NBUF
            for b in range(NBUF):
                wait_in(b)
                pltpu.async_copy(tab_hbm.at[idx_v[b].at[0]], gb_v[b], g_sem[b])
            for b in range(NBUF):
                pltpu.make_async_copy(
                    tab_hbm.at[idx_v[b].at[0]], gb_v[b], g_sem[b]
                ).wait()

                @plsc.parallel_loop(0, BSTRIPE // LANES, step=1, unroll=8)
                def bg_body(bg, b=b):
                    off = bg * LANES
                    bidx = iota16 + off
                    for d in range(dim):
                        vals = plsc.load_gather(
                            gb_v[b], [bidx, jnp.full((LANES,), d, jnp.int32)]
                        )
                        plsc.addupdate(
                            xb_v[b].at[0, d, pl.ds(off, LANES)], vals
                        )
                pltpu.async_copy(
                    xb_v[b],
                    out_hbm.at[pl.ds(h0 + b, 1), :, pl.ds(b0, BSTRIPE)],
                    out_sem[b],
                )
            for b in range(NBUF):
                wait_out(b)

                @pl.when(g < n_groups - 1)
                def _():
                    fire_in(b, h0 + NBUF + b)

            return carry

        lax.fori_loop(0, n_groups, group_body, 0, unroll=False)

    return k


def kernel(x, tta, embedding):
    nb, hist, d = x.shape
    vocab = embedding.shape[0]
    xt = jnp.transpose(x, (1, 2, 0))
    idxt = jnp.transpose((tta.astype(jnp.int32) - 1) % vocab, (1, 0))
    tabp = jnp.pad(embedding, ((0, 0), (0, PADW - d)))
    info = plsc.get_sparse_core_info()
    k = _tta_kernel(nb, hist, d, info.num_cores, info.num_subcores)
    outt = k(xt, idxt, tabp)
    return jnp.transpose(outt, (2, 0, 1))


# speedup vs baseline: 1.5088x; 1.5088x over previous
import functools

import jax
import jax.numpy as jnp
from jax import lax
from jax.experimental import pallas as pl
from jax.experimental.pallas import tpu as pltpu
from jax.experimental.pallas import tpu_sc as plsc

CHUNK = 64
NBUF = 4
LANES = 16
PADW = 128


def _tta_kernel(n_rows, dim, vocab, num_cores, num_subcores):
    n_workers = num_cores * num_subcores
    per_w = n_rows // n_workers
    n_chunks = per_w // CHUNK
    n_groups = n_chunks // NBUF
    mesh = plsc.VectorSubcoreMesh(core_axis_name="c", subcore_axis_name="s")

    @functools.partial(
        pl.kernel,
        mesh=mesh,
        out_type=jax.ShapeDtypeStruct((n_rows, dim), jnp.float32),
        scratch_types=(
            [pltpu.VMEM((1, CHUNK), jnp.int32)] * NBUF
            + [pltpu.VMEM((CHUNK, 64), jnp.float32)] * NBUF
            + [pltpu.VMEM((CHUNK, PADW), jnp.float32)] * NBUF
            + [pltpu.SemaphoreType.DMA] * (3 * NBUF)
        ),
    )
    def k(x_hbm, idx_hbm, tab_hbm, out_hbm, *scr):
        idx_v = scr[0:NBUF]
        xb_v = scr[NBUF : 2 * NBUF]
        gb_v = scr[2 * NBUF : 3 * NBUF]
        in_sem = scr[3 * NBUF : 4 * NBUF]
        g_sem = scr[4 * NBUF : 5 * NBUF]
        out_sem = scr[5 * NBUF : 6 * NBUF]

        wid = lax.axis_index("s") * num_cores + lax.axis_index("c")
        base = wid * per_w

        def fire_in(b, c):
            row0 = base + c * CHUNK
            pltpu.async_copy(
                idx_hbm.at[pl.ds(row0 // CHUNK, 1)], idx_v[b], in_sem[b]
            )
            pltpu.async_copy(x_hbm.at[pl.ds(row0, CHUNK)], xb_v[b], in_sem[b])

        def wait_in(b):
            pltpu.make_async_copy(
                idx_hbm.at[pl.ds(0, 1)], idx_v[b], in_sem[b]
            ).wait()
            pltpu.make_async_copy(
                x_hbm.at[pl.ds(0, CHUNK)], xb_v[b], in_sem[b]
            ).wait()

        def wait_out(b):
            pltpu.make_async_copy(
                xb_v[b], out_hbm.at[pl.ds(0, CHUNK)], out_sem[b]
            ).wait()

        for b in range(NBUF):
            fire_in(b, b)

        def group_body(g, carry):
            c0 = g * NBUF
            for b in range(NBUF):
                wait_in(b)
                for v in range(CHUNK // LANES):
                    t = idx_v[b][0, pl.ds(v * LANES, LANES)] - 1
                    t = jnp.where(t < 0, t + vocab, t)
                    idx_v[b][0, pl.ds(v * LANES, LANES)] = t
                pltpu.async_copy(
                    tab_hbm.at[idx_v[b].at[0]], gb_v[b], g_sem[b]
                )
            for b in range(NBUF):
                pltpu.make_async_copy(
                    tab_hbm.at[idx_v[b].at[0]], gb_v[b], g_sem[b]
                ).wait()

                @plsc.parallel_loop(0, CHUNK // 4, step=1, unroll=4)
                def add_rows(i, b=b):
                    r = i * 4
                    for rr in range(4):
                        for j in range(64 // LANES):
                            plsc.addupdate(
                                xb_v[b].at[r + rr, pl.ds(j * LANES, LANES)],
                                gb_v[b][r + rr, pl.ds(j * LANES, LANES)],
                            )

                row0 = base + (c0 + b) * CHUNK
                pltpu.async_copy(
                    xb_v[b], out_hbm.at[pl.ds(row0, CHUNK)], out_sem[b]
                )
            for b in range(NBUF):
                wait_out(b)

                @pl.when(g < n_groups - 1)
                def _():
                    fire_in(b, c0 + NBUF + b)

            return carry

        lax.fori_loop(0, n_groups, group_body, 0, unroll=False)

    return k


def kernel(x, tta, embedding):
    b, h, d = x.shape
    vocab = embedding.shape[0]
    n_rows = b * h
    x2 = x.reshape(n_rows, d)
    idx = tta.reshape(n_rows // CHUNK, CHUNK).astype(jnp.int32)
    tabp = jnp.pad(embedding, ((0, 0), (0, PADW - d)))
    info = plsc.get_sparse_core_info()
    k = _tta_kernel(n_rows, d, vocab, info.num_cores, info.num_subcores)
    out = k(x2, idx, tabp)
    return out.reshape(b, h, d)


# speedup vs baseline: 1.5517x; 1.0284x over previous
import functools

import jax
import jax.numpy as jnp
from jax import lax
from jax.experimental import pallas as pl
from jax.experimental.pallas import tpu as pltpu
from jax.experimental.pallas import tpu_sc as plsc

CHUNK = 64
NBUF = 5
LANES = 16
PADW = 128


def _tta_kernel(n_rows, dim, vocab, num_cores, num_subcores):
    n_workers = num_cores * num_subcores
    per_w = n_rows // n_workers
    n_chunks = per_w // CHUNK
    n_groups = n_chunks // NBUF
    mesh = plsc.VectorSubcoreMesh(core_axis_name="c", subcore_axis_name="s")

    @functools.partial(
        pl.kernel,
        mesh=mesh,
        out_type=jax.ShapeDtypeStruct((n_rows, dim), jnp.float32),
        scratch_types=(
            [pltpu.VMEM((1, CHUNK), jnp.int32)] * NBUF
            + [pltpu.VMEM((CHUNK, 64), jnp.float32)] * NBUF
            + [pltpu.VMEM((CHUNK, PADW), jnp.float32)] * NBUF
            + [pltpu.SemaphoreType.DMA] * (3 * NBUF)
        ),
    )
    def k(x_hbm, idx_hbm, tab_hbm, out_hbm, *scr):
        idx_v = scr[0:NBUF]
        xb_v = scr[NBUF : 2 * NBUF]
        gb_v = scr[2 * NBUF : 3 * NBUF]
        in_sem = scr[3 * NBUF : 4 * NBUF]
        g_sem = scr[4 * NBUF : 5 * NBUF]
        out_sem = scr[5 * NBUF : 6 * NBUF]

        wid = lax.axis_index("s") * num_cores + lax.axis_index("c")
        base = wid * per_w

        def fire_in(b, c):
            row0 = base + c * CHUNK
            pltpu.async_copy(
                idx_hbm.at[pl.ds(row0 // CHUNK, 1)], idx_v[b], in_sem[b]
            )
            pltpu.async_copy(x_hbm.at[pl.ds(row0, CHUNK)], xb_v[b], in_sem[b])

        def wait_in(b):
            pltpu.make_async_copy(
                idx_hbm.at[pl.ds(0, 1)], idx_v[b], in_sem[b]
            ).wait()
            pltpu.make_async_copy(
                x_hbm.at[pl.ds(0, CHUNK)], xb_v[b], in_sem[b]
            ).wait()

        def wait_out(b):
            pltpu.make_async_copy(
                xb_v[b], out_hbm.at[pl.ds(0, CHUNK)], out_sem[b]
            ).wait()

        for b in range(NBUF):
            fire_in(b, b)

        def group_body(g, carry):
            c0 = g * NBUF
            for b in range(NBUF):
                wait_in(b)
                for v in range(CHUNK // LANES):
                    t = idx_v[b][0, pl.ds(v * LANES, LANES)] - 1
                    t = jnp.where(t < 0, t + vocab, t)
                    idx_v[b][0, pl.ds(v * LANES, LANES)] = t
                pltpu.async_copy(
                    tab_hbm.at[idx_v[b].at[0]], gb_v[b], g_sem[b]
                )
            for b in range(NBUF):
                pltpu.make_async_copy(
                    tab_hbm.at[idx_v[b].at[0]], gb_v[b], g_sem[b]
                ).wait()

                @plsc.parallel_loop(0, CHUNK // 4, step=1, unroll=4)
                def add_rows(i, b=b):
                    r = i * 4
                    for rr in range(4):
                        for j in range(64 // LANES):
                            plsc.addupdate(
                                xb_v[b].at[r + rr, pl.ds(j * LANES, LANES)],
                                gb_v[b][r + rr, pl.ds(j * LANES, LANES)],
                            )

                row0 = base + (c0 + b) * CHUNK
                pltpu.async_copy(
                    xb_v[b], out_hbm.at[pl.ds(row0, CHUNK)], out_sem[b]
                )
            for b in range(NBUF):
                wait_out(b)

                @pl.when(g < n_groups - 1)
                def _():
                    fire_in(b, c0 + NBUF + b)

            return carry

        lax.fori_loop(0, n_groups, group_body, 0, unroll=False)

    return k


def kernel(x, tta, embedding):
    b, h, d = x.shape
    vocab = embedding.shape[0]
    n_rows = b * h
    x2 = x.reshape(n_rows, d)
    idx = tta.reshape(n_rows // CHUNK, CHUNK).astype(jnp.int32)
    tabp = jnp.pad(embedding, ((0, 0), (0, PADW - d)))
    info = plsc.get_sparse_core_info()
    k = _tta_kernel(n_rows, d, vocab, info.num_cores, info.num_subcores)
    out = k(x2, idx, tabp)
    return out.reshape(b, h, d)
